# baseline (speedup 1.0000x reference)
import functools

import jax
import jax.numpy as jnp
from jax import lax
from jax.experimental import pallas as pl
from jax.experimental.pallas import tpu as pltpu
from jax.experimental.pallas import tpu_sc as plsc

N = 10000
M = 16
F = 128
E = 16
B = 100
A = 100
NE = N * M
BN_BLK = 400
NBLK = N // BN_BLK
EDGE_BLK = BN_BLK * M

_NC, _NS = 2, 16
_NW = _NC * _NS
_CH = 128
_CPW = 40
_NE_PAD = _NW * _CPW * _CH
_NBUF = 5


def _softplus(x):
    return jnp.maximum(x, 0.0) + jnp.log1p(jnp.exp(-jnp.abs(x)))


def _sigmoid(x):
    return 0.5 * jnp.tanh(0.5 * x) + 0.5


def _sc_gather(table, idx2d):
    mesh = plsc.VectorSubcoreMesh(core_axis_name="c", subcore_axis_name="s")

    @functools.partial(
        pl.kernel,
        mesh=mesh,
        out_type=jax.ShapeDtypeStruct((_NE_PAD, F), jnp.float32),
        scratch_types=[
            pltpu.VMEM((_CPW, _CH), jnp.int32),
            pltpu.VMEM((_NBUF, _CH, F), jnp.float32),
        ] + [pltpu.SemaphoreType.DMA] * _NBUF,
    )
    def k(table_hbm, idx_hbm, out_hbm, idx_v, rows_v, *sems):
        wid = lax.axis_index("s") * _NC + lax.axis_index("c")
        pltpu.sync_copy(idx_hbm.at[pl.ds(wid * _CPW, _CPW)], idx_v)

        for b in range(_NBUF):
            pltpu.async_copy(table_hbm.at[idx_v.at[b]], rows_v.at[b], sems[b])

        def outer(j0, _):
            for b in range(_NBUF):
                j = j0 + b
                pltpu.make_async_copy(
                    table_hbm.at[idx_v.at[j]], rows_v.at[b], sems[b]).wait()
                off = (wid * _CPW + j) * _CH
                pltpu.sync_copy(rows_v.at[b], out_hbm.at[pl.ds(off, _CH)])
                pltpu.async_copy(
                    table_hbm.at[idx_v.at[j + _NBUF]], rows_v.at[b], sems[b])
            return ()

        lax.fori_loop(0, (_CPW - _NBUF) // _NBUF,
                      lambda t, c: outer(t * _NBUF, c), (), unroll=False)

        for b in range(_NBUF):
            j = _CPW - _NBUF + b
            pltpu.make_async_copy(
                table_hbm.at[idx_v.at[j]], rows_v.at[b], sems[b]).wait()
            off = (wid * _CPW + j) * _CH
            pltpu.sync_copy(rows_v.at[b], out_hbm.at[pl.ds(off, _CH)])

    return k(table, idx2d)


def _emb_kernel(fea_ref, tab_ref, out_ref):
    idx = fea_ref[...]
    kk = lax.broadcasted_iota(jnp.int32, (1, 128), 1)
    oh = (idx == kk).astype(jnp.float32)
    emb = jnp.dot(oh, tab_ref[...], preferred_element_type=jnp.float32,
                  precision=lax.Precision.HIGHEST)
    mu = jnp.mean(emb, axis=0, keepdims=True)
    var = jnp.mean((emb - mu) ** 2, axis=0, keepdims=True)
    out_ref[...] = (emb - mu) * lax.rsqrt(var + 1e-5)


def _emb(atom_fea, emb_pad):
    return pl.pallas_call(
        _emb_kernel,
        out_shape=jax.ShapeDtypeStruct((N, F), jnp.float32),
    )(atom_fea.reshape(N, 1).astype(jnp.int32), emb_pad)


def _stats1_kernel(atom_ref, gat_ref, nbr_ref, ws_ref, wn_ref, we_ref, b_ref,
                   out_ref, s1_ref, s2_ref):
    i = pl.program_id(0)

    @pl.when(i == 0)
    def _():
        s1_ref[...] = jnp.zeros_like(s1_ref)
        s2_ref[...] = jnp.zeros_like(s2_ref)

    g = jnp.dot(gat_ref[...], wn_ref[...], preferred_element_type=jnp.float32)
    g = g + jnp.dot(nbr_ref[...], we_ref[...],
                    preferred_element_type=jnp.float32)
    p = jnp.dot(atom_ref[...], ws_ref[...],
                preferred_element_type=jnp.float32) + b_ref[...]
    g3 = g.reshape(BN_BLK, M, 2 * F) + p[:, None, :]
    gf = g3.reshape(EDGE_BLK, 2 * F)
    ones = jnp.ones((1, EDGE_BLK), jnp.float32)
    s1_ref[...] += jnp.dot(ones, gf, preferred_element_type=jnp.float32,
                           precision=lax.Precision.HIGHEST)
    s2_ref[...] += jnp.dot(ones, gf * gf, preferred_element_type=jnp.float32,
                           precision=lax.Precision.HIGHEST)

    @pl.when(i == pl.num_programs(0) - 1)
    def _():
        out_ref[0:1, :] = s1_ref[...]
        out_ref[1:2, :] = s2_ref[...]


def _stats1(atom, gat, nbr2d, ws, wn, we, b2d):
    return pl.pallas_call(
        _stats1_kernel,
        grid=(NBLK,),
        in_specs=[
            pl.BlockSpec((BN_BLK, F), lambda i: (i, 0)),
            pl.BlockSpec((EDGE_BLK, F), lambda i: (i, 0)),
            pl.BlockSpec((EDGE_BLK, E), lambda i: (i, 0)),
            pl.BlockSpec((F, 2 * F), lambda i: (0, 0)),
            pl.BlockSpec((F, 2 * F), lambda i: (0, 0)),
            pl.BlockSpec((E, 2 * F), lambda i: (0, 0)),
            pl.BlockSpec((1, 2 * F), lambda i: (0, 0)),
        ],
        out_specs=pl.BlockSpec((2, 2 * F), lambda i: (0, 0)),
        out_shape=jax.ShapeDtypeStruct((2, 2 * F), jnp.float32),
        scratch_shapes=[
            pltpu.VMEM((1, 2 * F), jnp.float32),
            pltpu.VMEM((1, 2 * F), jnp.float32),
        ],
    )(atom, gat, nbr2d, ws, wn, we, b2d)


def _convB_kernel(atom_ref, gat_ref, nbr_ref, ws_ref, wn_ref, we_ref, b_ref,
                  st_ref, sum_ref, st2_ref, a1_ref, a2_ref):
    i = pl.program_id(0)

    @pl.when(i == 0)
    def _():
        a1_ref[...] = jnp.zeros_like(a1_ref)
        a2_ref[...] = jnp.zeros_like(a2_ref)

    mu = st_ref[0:1, :] / NE
    var = st_ref[1:2, :] / NE - mu * mu
    inv = lax.rsqrt(var + 1e-5)

    g = jnp.dot(gat_ref[...], wn_ref[...], preferred_element_type=jnp.float32)
    g = g + jnp.dot(nbr_ref[...], we_ref[...],
                    preferred_element_type=jnp.float32)
    p = jnp.dot(atom_ref[...], ws_ref[...],
                preferred_element_type=jnp.float32) + b_ref[...]
    g3 = g.reshape(BN_BLK, M, 2 * F) + p[:, None, :]
    g3 = (g3 - mu[None, :, :]) * inv[None, :, :]
    filt = _sigmoid(g3[:, :, :F])
    core = _softplus(g3[:, :, F:])
    s = jnp.sum(filt * core, axis=1)
    sum_ref[...] = s
    a1_ref[...] += jnp.sum(s, axis=0, keepdims=True)
    a2_ref[...] += jnp.sum(s * s, axis=0, keepdims=True)

    @pl.when(i == pl.num_programs(0) - 1)
    def _():
        st2_ref[0:1, :] = a1_ref[...]
        st2_ref[1:2, :] = a2_ref[...]


def _convB(atom, gat, nbr2d, ws, wn, we, b2d, st):
    return pl.pallas_call(
        _convB_kernel,
        grid=(NBLK,),
        in_specs=[
            pl.BlockSpec((BN_BLK, F), lambda i: (i, 0)),
            pl.BlockSpec((EDGE_BLK, F), lambda i: (i, 0)),
            pl.BlockSpec((EDGE_BLK, E), lambda i: (i, 0)),
            pl.BlockSpec((F, 2 * F), lambda i: (0, 0)),
            pl.BlockSpec((F, 2 * F), lambda i: (0, 0)),
            pl.BlockSpec((E, 2 * F), lambda i: (0, 0)),
            pl.BlockSpec((1, 2 * F), lambda i: (0, 0)),
            pl.BlockSpec((2, 2 * F), lambda i: (0, 0)),
        ],
        out_specs=[
            pl.BlockSpec((BN_BLK, F), lambda i: (i, 0)),
            pl.BlockSpec((2, F), lambda i: (0, 0)),
        ],
        out_shape=[
            jax.ShapeDtypeStruct((N, F), jnp.float32),
            jax.ShapeDtypeStruct((2, F), jnp.float32),
        ],
        scratch_shapes=[
            pltpu.VMEM((1, F), jnp.float32),
            pltpu.VMEM((1, F), jnp.float32),
        ],
    )(atom, gat, nbr2d, ws, wn, we, b2d, st)


def _update_kernel(atom_ref, sum_ref, st2_ref, out_ref):
    mu = st2_ref[0:1, :] / N
    var = st2_ref[1:2, :] / N - mu * mu
    inv = lax.rsqrt(var + 1e-5)
    out_ref[...] = _softplus(atom_ref[...] + (sum_ref[...] - mu) * inv)


def _update(atom, summed, st2):
    return pl.pallas_call(
        _update_kernel,
        out_shape=jax.ShapeDtypeStruct((N, F), jnp.float32),
    )(atom, summed, st2)


def _mean0(x):
    ones = jnp.ones((1, B), jnp.float32)
    return jnp.dot(ones, x, preferred_element_type=jnp.float32,
                   precision=lax.Precision.HIGHEST) * (1.0 / B)


def _head_kernel(atom_ref, ex_ref, wex_ref, bex_ref, wfc_ref, bfc_ref,
                 wout_ref, bout_ref, out_ref, task_ref):
    rows = lax.broadcasted_iota(jnp.int32, (B, N), 0)
    cols = lax.broadcasted_iota(jnp.int32, (B, N), 1)
    sel = (rows == cols // A).astype(jnp.float32)
    crys = jnp.dot(sel, atom_ref[...], preferred_element_type=jnp.float32,
                   precision=lax.Precision.HIGHEST) * (1.0 / A)

    ex = jnp.dot(ex_ref[...], wex_ref[...],
                 preferred_element_type=jnp.float32) + bex_ref[...]
    me = _mean0(ex)
    ve = _mean0((ex - me) ** 2)
    ex = _softplus((ex - me) * lax.rsqrt(ve + 1e-5))

    h = jnp.concatenate([crys, ex], axis=1)
    h = _softplus(jnp.dot(h, wfc_ref[...],
                          preferred_element_type=jnp.float32) + bfc_ref[...])
    mh = _mean0(h)
    vh = _mean0((h - mh) ** 2)
    h = (h - mh) * lax.rsqrt(vh + 1e-5)

    t = h + h
    mt = jnp.mean(t, axis=1, keepdims=True)
    vt = jnp.mean((t - mt) ** 2, axis=1, keepdims=True)
    task = (t - mt) * lax.rsqrt(vt + 1e-5)
    task_ref[...] = task
    out_ref[...] = jnp.dot(task, wout_ref[...],
                           preferred_element_type=jnp.float32) + bout_ref[...]


def _head(atom3, extra, wex, bex, wfc, bfc, wout, bout):
    return pl.pallas_call(
        _head_kernel,
        out_shape=[
            jax.ShapeDtypeStruct((B, 1), jnp.float32),
            jax.ShapeDtypeStruct((B, F), jnp.float32),
        ],
    )(atom3, extra, wex, bex.reshape(1, -1), wfc, bfc.reshape(1, -1),
      wout, bout.reshape(1, 1))


def kernel(atom_fea, nbr_fea_idx, nbr_fea, crystal_atom_idx, extra_fea,
           uni_idx, uni_count, emb_table, W_full1, b_full1, W_full2, b_full2,
           W_full3, b_full3, W_extra, b_extra, W_fc, b_fc, W_out, b_out):
    emb_pad = jnp.concatenate(
        [emb_table, jnp.zeros((128 - emb_table.shape[0], F),
                              dtype=jnp.float32)], axis=0)
    idx_flat = nbr_fea_idx.reshape(-1).astype(jnp.int32)
    idx2d = jnp.concatenate(
        [idx_flat, jnp.zeros((_NE_PAD - NE,), dtype=jnp.int32)]
    ).reshape(_NW * _CPW, _CH)
    nbr2d = nbr_fea.reshape(NE, E)

    atom = _emb(atom_fea, emb_pad)

    for (W, b) in ((W_full1, b_full1), (W_full2, b_full2),
                   (W_full3, b_full3)):
        ws, wn, we = W[:F], W[F:2 * F], W[2 * F:]
        b2d = b.reshape(1, 2 * F)
        gat = _sc_gather(atom, idx2d)[:NE]
        st = _stats1(atom, gat, nbr2d, ws, wn, we, b2d)
        summed, st2 = _convB(atom, gat, nbr2d, ws, wn, we, b2d, st)
        atom = _update(atom, summed, st2)

    out, task_fea = _head(atom, extra_fea, W_extra, b_extra, W_fc, b_fc,
                          W_out, b_out)
    return (out, task_fea)


# speedup vs baseline: 1.1790x; 1.1790x over previous
import functools

import jax
import jax.numpy as jnp
from jax import lax
from jax.experimental import pallas as pl
from jax.experimental.pallas import tpu as pltpu
from jax.experimental.pallas import tpu_sc as plsc

N = 10000
M = 16
F = 128
E = 16
B = 100
A = 100
NE = N * M
BN_BLK = 400
NBLK = N // BN_BLK
EDGE_BLK = BN_BLK * M

_NC, _NS = 2, 16
_NW = _NC * _NS
_CH = 128
_CPW = 40
_NE_PAD = _NW * _CPW * _CH
_NBUF = 5


def _softplus(x):
    return jnp.maximum(x, 0.0) + jnp.log1p(jnp.exp(-jnp.abs(x)))


def _sigmoid(x):
    return 0.5 * jnp.tanh(0.5 * x) + 0.5


def _sc_gather(table, idx2d):
    mesh = plsc.VectorSubcoreMesh(core_axis_name="c", subcore_axis_name="s")

    @functools.partial(
        pl.kernel,
        mesh=mesh,
        out_type=jax.ShapeDtypeStruct((_NE_PAD, F), jnp.float32),
        scratch_types=[
            pltpu.VMEM((_CPW, _CH), jnp.int32),
            pltpu.VMEM((_NBUF, _CH, F), jnp.float32),
        ] + [pltpu.SemaphoreType.DMA] * _NBUF,
    )
    def k(table_hbm, idx_hbm, out_hbm, idx_v, rows_v, *sems):
        wid = lax.axis_index("s") * _NC + lax.axis_index("c")
        pltpu.sync_copy(idx_hbm.at[pl.ds(wid * _CPW, _CPW)], idx_v)

        for b in range(_NBUF):
            pltpu.async_copy(table_hbm.at[idx_v.at[b]], rows_v.at[b], sems[b])

        def outer(j0, _):
            for b in range(_NBUF):
                j = j0 + b
                pltpu.make_async_copy(
                    table_hbm.at[idx_v.at[j]], rows_v.at[b], sems[b]).wait()
                off = (wid * _CPW + j) * _CH
                pltpu.sync_copy(rows_v.at[b], out_hbm.at[pl.ds(off, _CH)])
                pltpu.async_copy(
                    table_hbm.at[idx_v.at[j + _NBUF]], rows_v.at[b], sems[b])
            return ()

        lax.fori_loop(0, (_CPW - _NBUF) // _NBUF,
                      lambda t, c: outer(t * _NBUF, c), (), unroll=False)

        for b in range(_NBUF):
            j = _CPW - _NBUF + b
            pltpu.make_async_copy(
                table_hbm.at[idx_v.at[j]], rows_v.at[b], sems[b]).wait()
            off = (wid * _CPW + j) * _CH
            pltpu.sync_copy(rows_v.at[b], out_hbm.at[pl.ds(off, _CH)])

    return k(table, idx2d)


def _emb_kernel(fea_ref, tab_ref, out_ref):
    idx = fea_ref[...]
    kk = lax.broadcasted_iota(jnp.int32, (1, 128), 1)
    oh = (idx == kk).astype(jnp.float32)
    emb = jnp.dot(oh, tab_ref[...], preferred_element_type=jnp.float32,
                  precision=lax.Precision.HIGHEST)
    mu = jnp.mean(emb, axis=0, keepdims=True)
    var = jnp.mean((emb - mu) ** 2, axis=0, keepdims=True)
    out_ref[...] = (emb - mu) * lax.rsqrt(var + 1e-5)


def _emb(atom_fea, emb_pad):
    return pl.pallas_call(
        _emb_kernel,
        out_shape=jax.ShapeDtypeStruct((N, F), jnp.float32),
    )(atom_fea.reshape(N, 1).astype(jnp.int32), emb_pad)


def _stats1_kernel(atom_ref, gat_ref, nbr_ref, ws_ref, wn_ref, we_ref, b_ref,
                   out_ref, s1_ref, s2_ref):
    i = pl.program_id(0)

    @pl.when(i == 0)
    def _():
        s1_ref[...] = jnp.zeros_like(s1_ref)
        s2_ref[...] = jnp.zeros_like(s2_ref)

    g = jnp.dot(gat_ref[...], wn_ref[...], preferred_element_type=jnp.float32)
    g = g + jnp.dot(nbr_ref[...], we_ref[...],
                    preferred_element_type=jnp.float32)
    p = jnp.dot(atom_ref[...], ws_ref[...],
                preferred_element_type=jnp.float32) + b_ref[...]
    g3 = g.reshape(BN_BLK, M, 2 * F) + p[:, None, :]
    s1_ref[...] += jnp.sum(jnp.sum(g3, axis=1), axis=0, keepdims=True)
    s2_ref[...] += jnp.sum(jnp.sum(g3 * g3, axis=1), axis=0, keepdims=True)

    @pl.when(i == pl.num_programs(0) - 1)
    def _():
        out_ref[0:1, :] = s1_ref[...]
        out_ref[1:2, :] = s2_ref[...]


def _stats1(atom, gat, nbr2d, ws, wn, we, b2d):
    return pl.pallas_call(
        _stats1_kernel,
        grid=(NBLK,),
        in_specs=[
            pl.BlockSpec((BN_BLK, F), lambda i: (i, 0)),
            pl.BlockSpec((EDGE_BLK, F), lambda i: (i, 0)),
            pl.BlockSpec((EDGE_BLK, E), lambda i: (i, 0)),
            pl.BlockSpec((F, 2 * F), lambda i: (0, 0)),
            pl.BlockSpec((F, 2 * F), lambda i: (0, 0)),
            pl.BlockSpec((E, 2 * F), lambda i: (0, 0)),
            pl.BlockSpec((1, 2 * F), lambda i: (0, 0)),
        ],
        out_specs=pl.BlockSpec((2, 2 * F), lambda i: (0, 0)),
        out_shape=jax.ShapeDtypeStruct((2, 2 * F), jnp.float32),
        scratch_shapes=[
            pltpu.VMEM((1, 2 * F), jnp.float32),
            pltpu.VMEM((1, 2 * F), jnp.float32),
        ],
    )(atom, gat, nbr2d, ws, wn, we, b2d)


def _convB_kernel(atom_ref, gat_ref, nbr_ref, ws_ref, wn_ref, we_ref, b_ref,
                  st_ref, sum_ref, st2_ref, a1_ref, a2_ref):
    i = pl.program_id(0)

    @pl.when(i == 0)
    def _():
        a1_ref[...] = jnp.zeros_like(a1_ref)
        a2_ref[...] = jnp.zeros_like(a2_ref)

    mu = st_ref[0:1, :] / NE
    var = st_ref[1:2, :] / NE - mu * mu
    inv = lax.rsqrt(var + 1e-5)

    g = jnp.dot(gat_ref[...], wn_ref[...], preferred_element_type=jnp.float32)
    g = g + jnp.dot(nbr_ref[...], we_ref[...],
                    preferred_element_type=jnp.float32)
    p = jnp.dot(atom_ref[...], ws_ref[...],
                preferred_element_type=jnp.float32) + b_ref[...]
    g3 = g.reshape(BN_BLK, M, 2 * F) + p[:, None, :]
    g3 = (g3 - mu[None, :, :]) * inv[None, :, :]
    filt = _sigmoid(g3[:, :, :F])
    core = _softplus(g3[:, :, F:])
    s = jnp.sum(filt * core, axis=1)
    sum_ref[...] = s
    a1_ref[...] += jnp.sum(s, axis=0, keepdims=True)
    a2_ref[...] += jnp.sum(s * s, axis=0, keepdims=True)

    @pl.when(i == pl.num_programs(0) - 1)
    def _():
        st2_ref[0:1, :] = a1_ref[...]
        st2_ref[1:2, :] = a2_ref[...]


def _convB(atom, gat, nbr2d, ws, wn, we, b2d, st):
    return pl.pallas_call(
        _convB_kernel,
        grid=(NBLK,),
        in_specs=[
            pl.BlockSpec((BN_BLK, F), lambda i: (i, 0)),
            pl.BlockSpec((EDGE_BLK, F), lambda i: (i, 0)),
            pl.BlockSpec((EDGE_BLK, E), lambda i: (i, 0)),
            pl.BlockSpec((F, 2 * F), lambda i: (0, 0)),
            pl.BlockSpec((F, 2 * F), lambda i: (0, 0)),
            pl.BlockSpec((E, 2 * F), lambda i: (0, 0)),
            pl.BlockSpec((1, 2 * F), lambda i: (0, 0)),
            pl.BlockSpec((2, 2 * F), lambda i: (0, 0)),
        ],
        out_specs=[
            pl.BlockSpec((BN_BLK, F), lambda i: (i, 0)),
            pl.BlockSpec((2, F), lambda i: (0, 0)),
        ],
        out_shape=[
            jax.ShapeDtypeStruct((N, F), jnp.float32),
            jax.ShapeDtypeStruct((2, F), jnp.float32),
        ],
        scratch_shapes=[
            pltpu.VMEM((1, F), jnp.float32),
            pltpu.VMEM((1, F), jnp.float32),
        ],
    )(atom, gat, nbr2d, ws, wn, we, b2d, st)


def _update_kernel(atom_ref, sum_ref, st2_ref, out_ref):
    mu = st2_ref[0:1, :] / N
    var = st2_ref[1:2, :] / N - mu * mu
    inv = lax.rsqrt(var + 1e-5)
    out_ref[...] = _softplus(atom_ref[...] + (sum_ref[...] - mu) * inv)


def _update(atom, summed, st2):
    return pl.pallas_call(
        _update_kernel,
        out_shape=jax.ShapeDtypeStruct((N, F), jnp.float32),
    )(atom, summed, st2)


def _mean0(x):
    ones = jnp.ones((1, B), jnp.float32)
    return jnp.dot(ones, x, preferred_element_type=jnp.float32,
                   precision=lax.Precision.HIGHEST) * (1.0 / B)


def _head_kernel(atom_ref, ex_ref, wex_ref, bex_ref, wfc_ref, bfc_ref,
                 wout_ref, bout_ref, out_ref, task_ref):
    rows = lax.broadcasted_iota(jnp.int32, (B, N), 0)
    cols = lax.broadcasted_iota(jnp.int32, (B, N), 1)
    sel = (rows == cols // A).astype(jnp.float32)
    crys = jnp.dot(sel, atom_ref[...], preferred_element_type=jnp.float32,
                   precision=lax.Precision.HIGHEST) * (1.0 / A)

    ex = jnp.dot(ex_ref[...], wex_ref[...],
                 preferred_element_type=jnp.float32) + bex_ref[...]
    me = _mean0(ex)
    ve = _mean0((ex - me) ** 2)
    ex = _softplus((ex - me) * lax.rsqrt(ve + 1e-5))

    h = jnp.concatenate([crys, ex], axis=1)
    h = _softplus(jnp.dot(h, wfc_ref[...],
                          preferred_element_type=jnp.float32) + bfc_ref[...])
    mh = _mean0(h)
    vh = _mean0((h - mh) ** 2)
    h = (h - mh) * lax.rsqrt(vh + 1e-5)

    t = h + h
    mt = jnp.mean(t, axis=1, keepdims=True)
    vt = jnp.mean((t - mt) ** 2, axis=1, keepdims=True)
    task = (t - mt) * lax.rsqrt(vt + 1e-5)
    task_ref[...] = task
    out_ref[...] = jnp.dot(task, wout_ref[...],
                           preferred_element_type=jnp.float32) + bout_ref[...]


def _head(atom3, extra, wex, bex, wfc, bfc, wout, bout):
    return pl.pallas_call(
        _head_kernel,
        out_shape=[
            jax.ShapeDtypeStruct((B, 1), jnp.float32),
            jax.ShapeDtypeStruct((B, F), jnp.float32),
        ],
    )(atom3, extra, wex, bex.reshape(1, -1), wfc, bfc.reshape(1, -1),
      wout, bout.reshape(1, 1))


def kernel(atom_fea, nbr_fea_idx, nbr_fea, crystal_atom_idx, extra_fea,
           uni_idx, uni_count, emb_table, W_full1, b_full1, W_full2, b_full2,
           W_full3, b_full3, W_extra, b_extra, W_fc, b_fc, W_out, b_out):
    emb_pad = jnp.concatenate(
        [emb_table, jnp.zeros((128 - emb_table.shape[0], F),
                              dtype=jnp.float32)], axis=0)
    idx_flat = nbr_fea_idx.reshape(-1).astype(jnp.int32)
    idx2d = jnp.concatenate(
        [idx_flat, jnp.zeros((_NE_PAD - NE,), dtype=jnp.int32)]
    ).reshape(_NW * _CPW, _CH)
    nbr2d = nbr_fea.reshape(NE, E)

    atom = _emb(atom_fea, emb_pad)

    for (W, b) in ((W_full1, b_full1), (W_full2, b_full2),
                   (W_full3, b_full3)):
        ws, wn, we = W[:F], W[F:2 * F], W[2 * F:]
        b2d = b.reshape(1, 2 * F)
        gat = _sc_gather(atom, idx2d)[:NE]
        st = _stats1(atom, gat, nbr2d, ws, wn, we, b2d)
        summed, st2 = _convB(atom, gat, nbr2d, ws, wn, we, b2d, st)
        atom = _update(atom, summed, st2)

    out, task_fea = _head(atom, extra_fea, W_extra, b_extra, W_fc, b_fc,
                          W_out, b_out)
    return (out, task_fea)


# speedup vs baseline: 1.8550x; 1.5734x over previous
import functools

import jax
import jax.numpy as jnp
from jax import lax
from jax.experimental import pallas as pl
from jax.experimental.pallas import tpu as pltpu
from jax.experimental.pallas import tpu_sc as plsc

N = 10000
M = 16
F = 128
E = 16
B = 100
A = 100
NE = N * M
BN_BLK = 400
NBLK = N // BN_BLK
EDGE_BLK = BN_BLK * M

_NC, _NS = 2, 16
_NW = _NC * _NS
_CH = 128
_CPW = 40
_PW = NE // _NW
_NBUF = 5


def _softplus(x):
    return jnp.maximum(x, 0.0) + jnp.log1p(jnp.exp(-jnp.abs(x)))


def _sigmoid(x):
    return 0.5 * jnp.tanh(0.5 * x) + 0.5


def _sc_gather(table, idx2d):
    mesh = plsc.VectorSubcoreMesh(core_axis_name="c", subcore_axis_name="s")

    @functools.partial(
        pl.kernel,
        mesh=mesh,
        out_type=jax.ShapeDtypeStruct((NE, F), jnp.float32),
        scratch_types=[
            pltpu.VMEM((_CPW, _CH), jnp.int32),
            pltpu.VMEM((_NBUF, _CH, F), jnp.float32),
        ] + [pltpu.SemaphoreType.DMA] * _NBUF,
    )
    def k(table_hbm, idx_hbm, out_hbm, idx_v, rows_v, *sems):
        wid = lax.axis_index("s") * _NC + lax.axis_index("c")
        pltpu.sync_copy(idx_hbm.at[pl.ds(wid * _CPW, _CPW)], idx_v)

        def chunk_off(j):
            return wid * _PW + lax.min(j * _CH, _PW - _CH)

        for b in range(_NBUF):
            pltpu.async_copy(table_hbm.at[idx_v.at[b]], rows_v.at[b], sems[b])

        def outer(j0, _):
            for b in range(_NBUF):
                j = j0 + b
                pltpu.make_async_copy(
                    table_hbm.at[idx_v.at[j]], rows_v.at[b], sems[b]).wait()
                pltpu.sync_copy(rows_v.at[b],
                                out_hbm.at[pl.ds(chunk_off(j), _CH)])
                pltpu.async_copy(
                    table_hbm.at[idx_v.at[j + _NBUF]], rows_v.at[b], sems[b])
            return ()

        lax.fori_loop(0, (_CPW - _NBUF) // _NBUF,
                      lambda t, c: outer(t * _NBUF, c), (), unroll=False)

        for b in range(_NBUF):
            j = _CPW - _NBUF + b
            pltpu.make_async_copy(
                table_hbm.at[idx_v.at[j]], rows_v.at[b], sems[b]).wait()
            pltpu.sync_copy(rows_v.at[b],
                            out_hbm.at[pl.ds(chunk_off(j), _CH)])

    return k(table, idx2d)


def _emb_kernel(fea_ref, tab_ref, out_ref):
    idx = fea_ref[...]
    kk = lax.broadcasted_iota(jnp.int32, (1, 128), 1)
    oh = (idx == kk).astype(jnp.float32)
    emb = jnp.dot(oh, tab_ref[...], preferred_element_type=jnp.float32,
                  precision=lax.Precision.HIGHEST)
    mu = jnp.mean(emb, axis=0, keepdims=True)
    var = jnp.mean((emb - mu) ** 2, axis=0, keepdims=True)
    out_ref[...] = (emb - mu) * lax.rsqrt(var + 1e-5)


def _emb(atom_fea, emb_pad):
    return pl.pallas_call(
        _emb_kernel,
        out_shape=jax.ShapeDtypeStruct((N, F), jnp.float32),
    )(atom_fea.reshape(N, 1).astype(jnp.int32), emb_pad)


def _conv_kernel(atom_ref, gat_ref, nbr_ref, ws_ref, wn_ref, we_ref, b_ref,
                 sum_ref, st2_ref, s1_ref, s2_ref, a1_ref, a2_ref):
    ph = pl.program_id(0)
    i = pl.program_id(1)

    @pl.when((ph == 0) & (i == 0))
    def _():
        s1_ref[...] = jnp.zeros_like(s1_ref)
        s2_ref[...] = jnp.zeros_like(s2_ref)
        a1_ref[...] = jnp.zeros_like(a1_ref)
        a2_ref[...] = jnp.zeros_like(a2_ref)

    g = jnp.dot(gat_ref[...], wn_ref[...], preferred_element_type=jnp.float32)
    g = g + jnp.dot(nbr_ref[...], we_ref[...],
                    preferred_element_type=jnp.float32)
    p = jnp.dot(atom_ref[...], ws_ref[...],
                preferred_element_type=jnp.float32) + b_ref[...]
    g3 = g.reshape(BN_BLK, M, 2 * F) + p[:, None, :]

    @pl.when(ph == 0)
    def _():
        s1_ref[...] += jnp.sum(jnp.sum(g3, axis=1), axis=0, keepdims=True)
        s2_ref[...] += jnp.sum(jnp.sum(g3 * g3, axis=1), axis=0, keepdims=True)

    @pl.when(ph == 1)
    def _():
        mu = s1_ref[...] / NE
        var = s2_ref[...] / NE - mu * mu
        inv = lax.rsqrt(var + 1e-5)
        gn = (g3 - mu[None, :, :]) * inv[None, :, :]
        filt = _sigmoid(gn[:, :, :F])
        core = _softplus(gn[:, :, F:])
        sv = jnp.sum(filt * core, axis=1)
        sum_ref[...] = sv
        a1_ref[...] += jnp.sum(sv, axis=0, keepdims=True)
        a2_ref[...] += jnp.sum(sv * sv, axis=0, keepdims=True)

    @pl.when((ph == 1) & (i == pl.num_programs(1) - 1))
    def _():
        st2_ref[0:1, :] = a1_ref[...]
        st2_ref[1:2, :] = a2_ref[...]


def _conv(atom, gat, nbr2d, ws, wn, we, b2d):
    return pl.pallas_call(
        _conv_kernel,
        grid=(2, NBLK),
        in_specs=[
            pl.BlockSpec((BN_BLK, F), lambda ph, i: (i, 0)),
            pl.BlockSpec((EDGE_BLK, F), lambda ph, i: (i, 0)),
            pl.BlockSpec((EDGE_BLK, E), lambda ph, i: (i, 0)),
            pl.BlockSpec((F, 2 * F), lambda ph, i: (0, 0)),
            pl.BlockSpec((F, 2 * F), lambda ph, i: (0, 0)),
            pl.BlockSpec((E, 2 * F), lambda ph, i: (0, 0)),
            pl.BlockSpec((1, 2 * F), lambda ph, i: (0, 0)),
        ],
        out_specs=[
            pl.BlockSpec((BN_BLK, F), lambda ph, i: (i, 0)),
            pl.BlockSpec((2, F), lambda ph, i: (0, 0)),
        ],
        out_shape=[
            jax.ShapeDtypeStruct((N, F), jnp.float32),
            jax.ShapeDtypeStruct((2, F), jnp.float32),
        ],
        scratch_shapes=[
            pltpu.VMEM((1, 2 * F), jnp.float32),
            pltpu.VMEM((1, 2 * F), jnp.float32),
            pltpu.VMEM((1, F), jnp.float32),
            pltpu.VMEM((1, F), jnp.float32),
        ],
    )(atom, gat, nbr2d, ws, wn, we, b2d)


def _update_kernel(atom_ref, sum_ref, st2_ref, out_ref):
    mu = st2_ref[0:1, :] / N
    var = st2_ref[1:2, :] / N - mu * mu
    inv = lax.rsqrt(var + 1e-5)
    out_ref[...] = _softplus(atom_ref[...] + (sum_ref[...] - mu) * inv)


def _update(atom, summed, st2):
    return pl.pallas_call(
        _update_kernel,
        out_shape=jax.ShapeDtypeStruct((N, F), jnp.float32),
    )(atom, summed, st2)


def _mean0(x):
    ones = jnp.ones((1, B), jnp.float32)
    return jnp.dot(ones, x, preferred_element_type=jnp.float32,
                   precision=lax.Precision.HIGHEST) * (1.0 / B)


def _head_kernel(atom_ref, ex_ref, wex_ref, bex_ref, wfc_ref, bfc_ref,
                 wout_ref, bout_ref, out_ref, task_ref):
    rows = lax.broadcasted_iota(jnp.int32, (B, N), 0)
    cols = lax.broadcasted_iota(jnp.int32, (B, N), 1)
    sel = (rows == cols // A).astype(jnp.float32)
    crys = jnp.dot(sel, atom_ref[...], preferred_element_type=jnp.float32,
                   precision=lax.Precision.HIGHEST) * (1.0 / A)

    ex = jnp.dot(ex_ref[...], wex_ref[...],
                 preferred_element_type=jnp.float32) + bex_ref[...]
    me = _mean0(ex)
    ve = _mean0((ex - me) ** 2)
    ex = _softplus((ex - me) * lax.rsqrt(ve + 1e-5))

    h = jnp.concatenate([crys, ex], axis=1)
    h = _softplus(jnp.dot(h, wfc_ref[...],
                          preferred_element_type=jnp.float32) + bfc_ref[...])
    mh = _mean0(h)
    vh = _mean0((h - mh) ** 2)
    h = (h - mh) * lax.rsqrt(vh + 1e-5)

    t = h + h
    mt = jnp.mean(t, axis=1, keepdims=True)
    vt = jnp.mean((t - mt) ** 2, axis=1, keepdims=True)
    task = (t - mt) * lax.rsqrt(vt + 1e-5)
    task_ref[...] = task
    out_ref[...] = jnp.dot(task, wout_ref[...],
                           preferred_element_type=jnp.float32) + bout_ref[...]


def _head(atom3, extra, wex, bex, wfc, bfc, wout, bout):
    return pl.pallas_call(
        _head_kernel,
        out_shape=[
            jax.ShapeDtypeStruct((B, 1), jnp.float32),
            jax.ShapeDtypeStruct((B, F), jnp.float32),
        ],
    )(atom3, extra, wex, bex.reshape(1, -1), wfc, bfc.reshape(1, -1),
      wout, bout.reshape(1, 1))


def kernel(atom_fea, nbr_fea_idx, nbr_fea, crystal_atom_idx, extra_fea,
           uni_idx, uni_count, emb_table, W_full1, b_full1, W_full2, b_full2,
           W_full3, b_full3, W_extra, b_extra, W_fc, b_fc, W_out, b_out):
    emb_pad = jnp.concatenate(
        [emb_table, jnp.zeros((128 - emb_table.shape[0], F),
                              dtype=jnp.float32)], axis=0)
    idx_flat = nbr_fea_idx.reshape(-1).astype(jnp.int32)
    jj = jnp.minimum(jnp.arange(_CPW) * _CH, _PW - _CH)
    starts = (jnp.arange(_NW)[:, None] * _PW + jj[None, :]).reshape(-1)
    idx2d = idx_flat[starts[:, None] + jnp.arange(_CH)[None, :]]
    nbr2d = nbr_fea.reshape(NE, E)

    atom = _emb(atom_fea, emb_pad)

    for (W, b) in ((W_full1, b_full1), (W_full2, b_full2),
                   (W_full3, b_full3)):
        ws, wn, we = W[:F], W[F:2 * F], W[2 * F:]
        b2d = b.reshape(1, 2 * F)
        gat = _sc_gather(atom, idx2d)
        summed, st2 = _conv(atom, gat, nbr2d, ws, wn, we, b2d)
        atom = _update(atom, summed, st2)

    out, task_fea = _head(atom, extra_fea, W_extra, b_extra, W_fc, b_fc,
                          W_out, b_out)
    return (out, task_fea)


# speedup vs baseline: 1.9120x; 1.0307x over previous
import functools

import jax
import jax.numpy as jnp
from jax import lax
from jax.experimental import pallas as pl
from jax.experimental.pallas import tpu as pltpu
from jax.experimental.pallas import tpu_sc as plsc

N = 10000
M = 16
F = 128
E = 16
B = 100
A = 100
NE = N * M
BN_BLK = 400
NBLK = N // BN_BLK
EDGE_BLK = BN_BLK * M

_NC, _NS = 2, 16
_NW = _NC * _NS
_CH = 128
_CPW = 40
_PW = NE // _NW
_NBUF = 5


def _softplus(x):
    return jnp.maximum(x, 0.0) + jnp.log1p(jnp.exp(-jnp.abs(x)))


def _sigmoid(x):
    return 0.5 * jnp.tanh(0.5 * x) + 0.5


def _sc_gather(table, idx2d):
    mesh = plsc.VectorSubcoreMesh(core_axis_name="c", subcore_axis_name="s")

    @functools.partial(
        pl.kernel,
        mesh=mesh,
        out_type=jax.ShapeDtypeStruct((NE, F), jnp.float32),
        scratch_types=[
            pltpu.VMEM((_CPW, _CH), jnp.int32),
            pltpu.VMEM((_NBUF, _CH, F), jnp.float32),
        ] + [pltpu.SemaphoreType.DMA] * _NBUF,
    )
    def k(table_hbm, idx_hbm, out_hbm, idx_v, rows_v, *sems):
        wid = lax.axis_index("s") * _NC + lax.axis_index("c")
        pltpu.sync_copy(idx_hbm.at[pl.ds(wid * _CPW, _CPW)], idx_v)

        def chunk_off(j):
            return wid * _PW + lax.min(j * _CH, _PW - _CH)

        for b in range(_NBUF):
            pltpu.async_copy(table_hbm.at[idx_v.at[b]], rows_v.at[b], sems[b])

        def outer(j0, _):
            for b in range(_NBUF):
                j = j0 + b
                pltpu.make_async_copy(
                    table_hbm.at[idx_v.at[j]], rows_v.at[b], sems[b]).wait()
                pltpu.sync_copy(rows_v.at[b],
                                out_hbm.at[pl.ds(chunk_off(j), _CH)])
                pltpu.async_copy(
                    table_hbm.at[idx_v.at[j + _NBUF]], rows_v.at[b], sems[b])
            return ()

        lax.fori_loop(0, (_CPW - _NBUF) // _NBUF,
                      lambda t, c: outer(t * _NBUF, c), (), unroll=False)

        for b in range(_NBUF):
            j = _CPW - _NBUF + b
            pltpu.make_async_copy(
                table_hbm.at[idx_v.at[j]], rows_v.at[b], sems[b]).wait()
            pltpu.sync_copy(rows_v.at[b],
                            out_hbm.at[pl.ds(chunk_off(j), _CH)])

    return k(table, idx2d)


def _emb_kernel(fea_ref, tab_ref, out_ref):
    idx = fea_ref[...]
    kk = lax.broadcasted_iota(jnp.int32, (1, 128), 1)
    oh = (idx == kk).astype(jnp.float32)
    emb = jnp.dot(oh, tab_ref[...], preferred_element_type=jnp.float32,
                  precision=lax.Precision.HIGHEST)
    mu = jnp.mean(emb, axis=0, keepdims=True)
    var = jnp.mean((emb - mu) ** 2, axis=0, keepdims=True)
    out_ref[...] = (emb - mu) * lax.rsqrt(var + 1e-5)


def _emb(atom_fea, emb_pad):
    return pl.pallas_call(
        _emb_kernel,
        out_shape=jax.ShapeDtypeStruct((N, F), jnp.float32),
    )(atom_fea.reshape(N, 1).astype(jnp.int32), emb_pad)


def _stats1_kernel(atom_ref, gat_ref, nbr_ref, ws_ref, wn_ref, we_ref, b_ref,
                   out_ref, s1_ref, s2_ref):
    i = pl.program_id(0)

    @pl.when(i == 0)
    def _():
        s1_ref[...] = jnp.zeros_like(s1_ref)
        s2_ref[...] = jnp.zeros_like(s2_ref)

    g = jnp.dot(gat_ref[...], wn_ref[...], preferred_element_type=jnp.float32)
    g = g + jnp.dot(nbr_ref[...], we_ref[...],
                    preferred_element_type=jnp.float32)
    p = jnp.dot(atom_ref[...], ws_ref[...],
                preferred_element_type=jnp.float32) + b_ref[...]
    g3 = g.reshape(BN_BLK, M, 2 * F) + p[:, None, :]
    s1_ref[...] += jnp.sum(jnp.sum(g3, axis=1), axis=0, keepdims=True)
    s2_ref[...] += jnp.sum(jnp.sum(g3 * g3, axis=1), axis=0, keepdims=True)

    @pl.when(i == pl.num_programs(0) - 1)
    def _():
        out_ref[0:1, :] = s1_ref[...]
        out_ref[1:2, :] = s2_ref[...]


def _stats1(atom, gat, nbr2d, ws, wn, we, b2d):
    return pl.pallas_call(
        _stats1_kernel,
        grid=(NBLK,),
        in_specs=[
            pl.BlockSpec((BN_BLK, F), lambda i: (i, 0)),
            pl.BlockSpec((EDGE_BLK, F), lambda i: (i, 0)),
            pl.BlockSpec((EDGE_BLK, E), lambda i: (i, 0)),
            pl.BlockSpec((F, 2 * F), lambda i: (0, 0)),
            pl.BlockSpec((F, 2 * F), lambda i: (0, 0)),
            pl.BlockSpec((E, 2 * F), lambda i: (0, 0)),
            pl.BlockSpec((1, 2 * F), lambda i: (0, 0)),
        ],
        out_specs=pl.BlockSpec((2, 2 * F), lambda i: (0, 0)),
        out_shape=jax.ShapeDtypeStruct((2, 2 * F), jnp.float32),
        scratch_shapes=[
            pltpu.VMEM((1, 2 * F), jnp.float32),
            pltpu.VMEM((1, 2 * F), jnp.float32),
        ],
    )(atom, gat, nbr2d, ws, wn, we, b2d)


def _convB_kernel(atom_ref, gat_ref, nbr_ref, ws_ref, wn_ref, we_ref, b_ref,
                  st_ref, sum_ref, st2_ref, a1_ref, a2_ref):
    i = pl.program_id(0)

    @pl.when(i == 0)
    def _():
        a1_ref[...] = jnp.zeros_like(a1_ref)
        a2_ref[...] = jnp.zeros_like(a2_ref)

    mu = st_ref[0:1, :] / NE
    var = st_ref[1:2, :] / NE - mu * mu
    inv = lax.rsqrt(var + 1e-5)

    g = jnp.dot(gat_ref[...], wn_ref[...], preferred_element_type=jnp.float32)
    g = g + jnp.dot(nbr_ref[...], we_ref[...],
                    preferred_element_type=jnp.float32)
    p = jnp.dot(atom_ref[...], ws_ref[...],
                preferred_element_type=jnp.float32) + b_ref[...]
    g3 = g.reshape(BN_BLK, M, 2 * F) + p[:, None, :]
    g3 = (g3 - mu[None, :, :]) * inv[None, :, :]
    filt = _sigmoid(g3[:, :, :F])
    core = _softplus(g3[:, :, F:])
    s = jnp.sum(filt * core, axis=1)
    sum_ref[...] = s
    a1_ref[...] += jnp.sum(s, axis=0, keepdims=True)
    a2_ref[...] += jnp.sum(s * s, axis=0, keepdims=True)

    @pl.when(i == pl.num_programs(0) - 1)
    def _():
        st2_ref[0:1, :] = a1_ref[...]
        st2_ref[1:2, :] = a2_ref[...]


def _convB(atom, gat, nbr2d, ws, wn, we, b2d, st):
    return pl.pallas_call(
        _convB_kernel,
        grid=(NBLK,),
        in_specs=[
            pl.BlockSpec((BN_BLK, F), lambda i: (i, 0)),
            pl.BlockSpec((EDGE_BLK, F), lambda i: (i, 0)),
            pl.BlockSpec((EDGE_BLK, E), lambda i: (i, 0)),
            pl.BlockSpec((F, 2 * F), lambda i: (0, 0)),
            pl.BlockSpec((F, 2 * F), lambda i: (0, 0)),
            pl.BlockSpec((E, 2 * F), lambda i: (0, 0)),
            pl.BlockSpec((1, 2 * F), lambda i: (0, 0)),
            pl.BlockSpec((2, 2 * F), lambda i: (0, 0)),
        ],
        out_specs=[
            pl.BlockSpec((BN_BLK, F), lambda i: (i, 0)),
            pl.BlockSpec((2, F), lambda i: (0, 0)),
        ],
        out_shape=[
            jax.ShapeDtypeStruct((N, F), jnp.float32),
            jax.ShapeDtypeStruct((2, F), jnp.float32),
        ],
        scratch_shapes=[
            pltpu.VMEM((1, F), jnp.float32),
            pltpu.VMEM((1, F), jnp.float32),
        ],
    )(atom, gat, nbr2d, ws, wn, we, b2d, st)


def _update_kernel(atom_ref, sum_ref, st2_ref, out_ref):
    mu = st2_ref[0:1, :] / N
    var = st2_ref[1:2, :] / N - mu * mu
    inv = lax.rsqrt(var + 1e-5)
    out_ref[...] = _softplus(atom_ref[...] + (sum_ref[...] - mu) * inv)


def _update(atom, summed, st2):
    return pl.pallas_call(
        _update_kernel,
        out_shape=jax.ShapeDtypeStruct((N, F), jnp.float32),
    )(atom, summed, st2)


def _mean0(x):
    ones = jnp.ones((1, B), jnp.float32)
    return jnp.dot(ones, x, preferred_element_type=jnp.float32,
                   precision=lax.Precision.HIGHEST) * (1.0 / B)


def _head_kernel(atom_ref, ex_ref, wex_ref, bex_ref, wfc_ref, bfc_ref,
                 wout_ref, bout_ref, out_ref, task_ref):
    rows = lax.broadcasted_iota(jnp.int32, (B, N), 0)
    cols = lax.broadcasted_iota(jnp.int32, (B, N), 1)
    sel = (rows == cols // A).astype(jnp.float32)
    crys = jnp.dot(sel, atom_ref[...], preferred_element_type=jnp.float32,
                   precision=lax.Precision.HIGHEST) * (1.0 / A)

    ex = jnp.dot(ex_ref[...], wex_ref[...],
                 preferred_element_type=jnp.float32) + bex_ref[...]
    me = _mean0(ex)
    ve = _mean0((ex - me) ** 2)
    ex = _softplus((ex - me) * lax.rsqrt(ve + 1e-5))

    h = jnp.concatenate([crys, ex], axis=1)
    h = _softplus(jnp.dot(h, wfc_ref[...],
                          preferred_element_type=jnp.float32) + bfc_ref[...])
    mh = _mean0(h)
    vh = _mean0((h - mh) ** 2)
    h = (h - mh) * lax.rsqrt(vh + 1e-5)

    t = h + h
    mt = jnp.mean(t, axis=1, keepdims=True)
    vt = jnp.mean((t - mt) ** 2, axis=1, keepdims=True)
    task = (t - mt) * lax.rsqrt(vt + 1e-5)
    task_ref[...] = task
    out_ref[...] = jnp.dot(task, wout_ref[...],
                           preferred_element_type=jnp.float32) + bout_ref[...]


def _head(atom3, extra, wex, bex, wfc, bfc, wout, bout):
    return pl.pallas_call(
        _head_kernel,
        out_shape=[
            jax.ShapeDtypeStruct((B, 1), jnp.float32),
            jax.ShapeDtypeStruct((B, F), jnp.float32),
        ],
    )(atom3, extra, wex, bex.reshape(1, -1), wfc, bfc.reshape(1, -1),
      wout, bout.reshape(1, 1))


def kernel(atom_fea, nbr_fea_idx, nbr_fea, crystal_atom_idx, extra_fea,
           uni_idx, uni_count, emb_table, W_full1, b_full1, W_full2, b_full2,
           W_full3, b_full3, W_extra, b_extra, W_fc, b_fc, W_out, b_out):
    emb_pad = jnp.concatenate(
        [emb_table, jnp.zeros((128 - emb_table.shape[0], F),
                              dtype=jnp.float32)], axis=0)
    idx_flat = nbr_fea_idx.reshape(-1).astype(jnp.int32)
    jj = jnp.minimum(jnp.arange(_CPW) * _CH, _PW - _CH)
    starts = (jnp.arange(_NW)[:, None] * _PW + jj[None, :]).reshape(-1)
    idx2d = idx_flat[starts[:, None] + jnp.arange(_CH)[None, :]]
    nbr2d = nbr_fea.reshape(NE, E)

    atom = _emb(atom_fea, emb_pad)

    for (W, b) in ((W_full1, b_full1), (W_full2, b_full2),
                   (W_full3, b_full3)):
        ws, wn, we = W[:F], W[F:2 * F], W[2 * F:]
        b2d = b.reshape(1, 2 * F)
        gat = _sc_gather(atom, idx2d)
        st = _stats1(atom, gat, nbr2d, ws, wn, we, b2d)
        summed, st2 = _convB(atom, gat, nbr2d, ws, wn, we, b2d, st)
        atom = _update(atom, summed, st2)

    out, task_fea = _head(atom, extra_fea, W_extra, b_extra, W_fc, b_fc,
                          W_out, b_out)
    return (out, task_fea)
